# baseline (speedup 1.0000x reference)
import functools
import math

import jax
import jax.numpy as jnp
from jax import lax
from jax.experimental import pallas as pl
from jax.experimental.pallas import tpu as pltpu
from jax.experimental.pallas import tpu_sc as plsc

_DEPTH = 12
_N_LEVELS = _DEPTH + 1
_SPLIT = 7
_N_TOP = 2 ** _SPLIT - 1
_N_TOP_PAD = 2 ** _SPLIT
_D = 1024
_BT = 512


def _tc_shallow_body(x_ref, w1t_ref, w2t_ref, partial_ref, nodes_ref):
    x = x_ref[...]
    w1t = w1t_ref[...]
    scores = jax.lax.dot_general(
        x, w1t, (((1,), (1,)), ((), ())),
        precision=lax.Precision.HIGHEST,
        preferred_element_type=jnp.float32)
    iota_n = lax.broadcasted_iota(jnp.int32, (_BT, _N_TOP_PAD), 1)
    cur = jnp.zeros((_BT, 1), dtype=jnp.int32)
    g_acc = jnp.zeros((_BT, _N_TOP_PAD), dtype=jnp.float32)
    for _ in range(_SPLIT):
        onehot = iota_n == cur
        logit = jnp.sum(jnp.where(onehot, scores, 0.0), axis=1,
                        keepdims=True)
        g = 0.5 * logit * (1.0 + lax.erf(logit * (1.0 / math.sqrt(2.0))))
        g_acc = g_acc + jnp.where(onehot, g, 0.0)
        cur = 2 * cur + 1 + (logit >= 0.0).astype(jnp.int32)
    w2t = w2t_ref[...]
    gh = g_acc.astype(jnp.bfloat16)
    gl = (g_acc - gh.astype(jnp.float32)).astype(jnp.bfloat16)
    wh = w2t.astype(jnp.bfloat16)
    wl = (w2t - wh.astype(jnp.float32)).astype(jnp.bfloat16)

    def _mm(a, bmat):
        return jax.lax.dot_general(
            a, bmat, (((1,), (0,)), ((), ())),
            preferred_element_type=jnp.float32)

    partial_ref[...] = _mm(gh, wh) + (_mm(gh, wl) + _mm(gl, wh))
    nodes_ref[...] = cur


def _tc_shallow(x, w1t, w2t):
    b = x.shape[0]
    grid = b // _BT
    partial, nodes = pl.pallas_call(
        _tc_shallow_body,
        grid=(grid,),
        in_specs=[
            pl.BlockSpec((_BT, _D), lambda i: (i, 0)),
            pl.BlockSpec((_N_TOP_PAD, _D), lambda i: (0, 0)),
            pl.BlockSpec((_N_TOP_PAD, _D), lambda i: (0, 0)),
        ],
        out_specs=[
            pl.BlockSpec((_BT, _D), lambda i: (i, 0)),
            pl.BlockSpec((_BT, 1), lambda i: (i, 0)),
        ],
        out_shape=[
            jax.ShapeDtypeStruct((b, _D), jnp.float32),
            jax.ShapeDtypeStruct((b, 1), jnp.int32),
        ],
    )(x, w1t, w2t)
    return partial, nodes.reshape(b)


_NC = 2
_NS = 16
_NW = _NC * _NS
_L = 16
_G = 16
_N_DEEP = _N_LEVELS - _SPLIT


def _sc_gelu(lg):
    z = lg * 0.7071067811865476
    a = jnp.abs(z)
    t = 1.0 / (1.0 + 0.3275911 * a)
    poly = ((((1.061405429 * t - 1.453152027) * t + 1.421413741) * t
             - 0.284496736) * t + 0.254829592) * t
    erf_a = 1.0 - poly * jnp.exp(-(a * a))
    erf_z = jnp.where(z < 0.0, -erf_a, erf_a)
    return 0.5 * lg * (1.0 + erf_z)


def _sc_dots(xb, w1b):
    nchunks = _D // _L

    def dot_body(j, accs):
        off = j * _L
        return tuple(
            accs[t] + xb[t, pl.ds(off, _L)] * w1b[t, pl.ds(off, _L)]
            for t in range(_G))

    accs = lax.fori_loop(
        0, nchunks, dot_body,
        tuple(jnp.zeros((_L,), jnp.float32) for _ in range(_G)))
    lane = lax.iota(jnp.int32, _L)
    lg = jnp.zeros((_L,), jnp.float32)
    for t in range(_G):
        lg = jnp.where(lane == t, jnp.sum(accs[t]), lg)
    return lg


def _sc_accum(accb, w2b, g):
    nblk = _D // (2 * _L)

    def body(j, _c):
        off = 2 * _L * j
        for t in range(_G):
            raw = plsc.bitcast(w2b[t, pl.ds(_L * j, _L)], jnp.bfloat16)
            pa, pb = plsc.unpack(raw, format=plsc.PackFormat.INTERLEAVED)
            accb[t, pl.ds(off, _L)] = accb[t, pl.ds(off, _L)] + g[t] * pa
            accb[t, pl.ds(off + _L, _L)] = (
                accb[t, pl.ds(off + _L, _L)] + g[t] * pb)
        return 0

    lax.fori_loop(0, nblk, body, 0)


def _sc_deep_body(x_hbm, w1_hbm, w2_hbm, part_hbm, nodes_hbm, out_hbm,
                  xbA, xbB, accbA, accbB, w1bA, w1bB, w2bA, w2bB,
                  idxA0, idxA1, idxB0, idxB1, semA, semB, semA2, semB2):
    wid = lax.axis_index("s") * _NC + lax.axis_index("c")
    tok_per_w = x_hbm.shape[0] // _NW
    n_pairs = tok_per_w // (2 * _G)

    def pair(pi, _):
        baseA = wid * tok_per_w + pi * (2 * _G)
        baseB = baseA + _G
        pltpu.sync_copy(x_hbm.at[pl.ds(baseA, _G)], xbA)
        pltpu.sync_copy(part_hbm.at[pl.ds(baseA, _G)], accbA)
        pltpu.sync_copy(nodes_hbm.at[pl.ds(baseA, _G)], idxA0)
        pltpu.sync_copy(x_hbm.at[pl.ds(baseB, _G)], xbB)
        pltpu.sync_copy(part_hbm.at[pl.ds(baseB, _G)], accbB)
        pltpu.sync_copy(nodes_hbm.at[pl.ds(baseB, _G)], idxB0)
        ibA = (idxA0, idxA1)
        ibB = (idxB0, idxB1)
        cpA = pltpu.async_copy(w1_hbm.at[idxA0], w1bA, semA)
        cpB = pltpu.async_copy(w1_hbm.at[idxB0], w1bB, semB)
        for lvl in range(_N_DEEP):
            curA, nxtA = ibA[lvl % 2], ibA[(lvl + 1) % 2]
            curB, nxtB = ibB[lvl % 2], ibB[(lvl + 1) % 2]
            cpA.wait()
            lgA = _sc_dots(xbA, w1bA)
            gA = _sc_gelu(lgA)
            nxtA[...] = 2 * curA[...] + 1 + (lgA >= 0.0).astype(jnp.int32)
            w2cpA = pltpu.async_copy(w2_hbm.at[curA], w2bA, semA2)
            if lvl < _N_DEEP - 1:
                cpA = pltpu.async_copy(w1_hbm.at[nxtA], w1bA, semA)
            cpB.wait()
            lgB = _sc_dots(xbB, w1bB)
            gB = _sc_gelu(lgB)
            nxtB[...] = 2 * curB[...] + 1 + (lgB >= 0.0).astype(jnp.int32)
            w2cpB = pltpu.async_copy(w2_hbm.at[curB], w2bB, semB2)
            if lvl < _N_DEEP - 1:
                cpB = pltpu.async_copy(w1_hbm.at[nxtB], w1bB, semB)
            w2cpA.wait()
            _sc_accum(accbA, w2bA, gA)
            w2cpB.wait()
            _sc_accum(accbB, w2bB, gB)
        pltpu.sync_copy(accbA, out_hbm.at[pl.ds(baseA, _G)])
        pltpu.sync_copy(accbB, out_hbm.at[pl.ds(baseB, _G)])
        return 0

    lax.fori_loop(0, n_pairs, pair, 0)


def _sc_deep(x, w1s, w2p, partial, nodes7):
    b = x.shape[0]
    fn = pl.kernel(
        _sc_deep_body,
        mesh=plsc.VectorSubcoreMesh(core_axis_name="c", subcore_axis_name="s"),
        compiler_params=pltpu.CompilerParams(needs_layout_passes=False),
        out_type=jax.ShapeDtypeStruct((b, _D), jnp.float32),
        scratch_types=[
            pltpu.VMEM((_G, _D), jnp.float32),
            pltpu.VMEM((_G, _D), jnp.float32),
            pltpu.VMEM((_G, _D), jnp.float32),
            pltpu.VMEM((_G, _D), jnp.float32),
            pltpu.VMEM((_G, _D), jnp.float32),
            pltpu.VMEM((_G, _D), jnp.float32),
            pltpu.VMEM((_G, _D // 2), jnp.int32),
            pltpu.VMEM((_G, _D // 2), jnp.int32),
            pltpu.VMEM((_G,), jnp.int32),
            pltpu.VMEM((_G,), jnp.int32),
            pltpu.VMEM((_G,), jnp.int32),
            pltpu.VMEM((_G,), jnp.int32),
            pltpu.SemaphoreType.DMA,
            pltpu.SemaphoreType.DMA,
            pltpu.SemaphoreType.DMA,
            pltpu.SemaphoreType.DMA,
        ],
    )
    return fn(x, w1s, w2p, partial, nodes7)


def kernel(x, w1s, w2s):
    w1t = jnp.concatenate(
        [w1s[:_N_TOP], jnp.zeros((_N_TOP_PAD - _N_TOP, _D), jnp.float32)], 0)
    w2t = jnp.concatenate(
        [w2s[:_N_TOP], jnp.zeros((_N_TOP_PAD - _N_TOP, _D), jnp.float32)], 0)
    w2p = (w2s.astype(jnp.bfloat16)
           .reshape(-1, _D // (2 * _L), 2, _L)
           .swapaxes(2, 3)
           .reshape(-1, _D // 2, 2))
    w2p = jax.lax.bitcast_convert_type(w2p, jnp.int32)
    partial, nodes7 = _tc_shallow(x, w1t, w2t)
    out = _sc_deep(x, w1s, w2p, partial, nodes7)
    return out[:, None, :]


# speedup vs baseline: 1.7581x; 1.7581x over previous
import functools
import math

import jax
import jax.numpy as jnp
from jax import lax
from jax.experimental import pallas as pl
from jax.experimental.pallas import tpu as pltpu
from jax.experimental.pallas import tpu_sc as plsc

_DEPTH = 12
_N_LEVELS = _DEPTH + 1
_SPLIT = 7
_N_TOP = 2 ** _SPLIT - 1
_N_TOP_PAD = 2 ** _SPLIT
_D = 1024
_BT = 512


def _tc_shallow_body(x_ref, w1t_ref, w2t_ref, partial_ref, nodes_ref):
    x = x_ref[...]
    w1t = w1t_ref[...]
    scores = jax.lax.dot_general(
        x, w1t, (((1,), (1,)), ((), ())),
        precision=lax.Precision.HIGHEST,
        preferred_element_type=jnp.float32)
    iota_n = lax.broadcasted_iota(jnp.int32, (_BT, _N_TOP_PAD), 1)
    cur = jnp.zeros((_BT, 1), dtype=jnp.int32)
    g_acc = jnp.zeros((_BT, _N_TOP_PAD), dtype=jnp.float32)
    for _ in range(_SPLIT):
        onehot = iota_n == cur
        logit = jnp.sum(jnp.where(onehot, scores, 0.0), axis=1,
                        keepdims=True)
        g = 0.5 * logit * (1.0 + lax.erf(logit * (1.0 / math.sqrt(2.0))))
        g_acc = g_acc + jnp.where(onehot, g, 0.0)
        cur = 2 * cur + 1 + (logit >= 0.0).astype(jnp.int32)
    w2t = w2t_ref[...]
    gh = g_acc.astype(jnp.bfloat16)
    gl = (g_acc - gh.astype(jnp.float32)).astype(jnp.bfloat16)
    wh = w2t.astype(jnp.bfloat16)
    wl = (w2t - wh.astype(jnp.float32)).astype(jnp.bfloat16)

    def _mm(a, bmat):
        return jax.lax.dot_general(
            a, bmat, (((1,), (0,)), ((), ())),
            preferred_element_type=jnp.float32)

    partial_ref[...] = _mm(gh, wh) + (_mm(gh, wl) + _mm(gl, wh))
    nodes_ref[...] = cur


def _tc_shallow(x, w1t, w2t):
    b = x.shape[0]
    grid = b // _BT
    partial, nodes = pl.pallas_call(
        _tc_shallow_body,
        grid=(grid,),
        in_specs=[
            pl.BlockSpec((_BT, _D), lambda i: (i, 0)),
            pl.BlockSpec((_N_TOP_PAD, _D), lambda i: (0, 0)),
            pl.BlockSpec((_N_TOP_PAD, _D), lambda i: (0, 0)),
        ],
        out_specs=[
            pl.BlockSpec((_BT, _D), lambda i: (i, 0)),
            pl.BlockSpec((_BT, 1), lambda i: (i, 0)),
        ],
        out_shape=[
            jax.ShapeDtypeStruct((b, _D), jnp.float32),
            jax.ShapeDtypeStruct((b, 1), jnp.int32),
        ],
    )(x, w1t, w2t)
    return partial, nodes.reshape(b)


_NC = 2
_NS = 16
_NW = _NC * _NS
_L = 16
_G = 16
_N_DEEP = _N_LEVELS - _SPLIT


def _sc_gelu(lg):
    z = lg * 0.7071067811865476
    a = jnp.abs(z)
    t = 1.0 / (1.0 + 0.3275911 * a)
    poly = ((((1.061405429 * t - 1.453152027) * t + 1.421413741) * t
             - 0.284496736) * t + 0.254829592) * t
    erf_a = 1.0 - poly * jnp.exp(-(a * a))
    erf_z = jnp.where(z < 0.0, -erf_a, erf_a)
    return 0.5 * lg * (1.0 + erf_z)


def _sc_dots(xb, w1b):
    nchunks = _D // _L

    def dot_body(j, accs):
        off = j * _L
        return tuple(
            accs[t] + xb[t, pl.ds(off, _L)] * w1b[t, pl.ds(off, _L)]
            for t in range(_G))

    accs = lax.fori_loop(
        0, nchunks, dot_body,
        tuple(jnp.zeros((_L,), jnp.float32) for _ in range(_G)))
    lane = lax.iota(jnp.int32, _L)
    lg = jnp.zeros((_L,), jnp.float32)
    for t in range(_G):
        lg = jnp.where(lane == t, jnp.sum(accs[t]), lg)
    return lg


def _sc_accum(accb, rowb, g):
    nchunks = _D // _L

    def acc_body(j, _c):
        off = j * _L
        for t in range(_G):
            accb[t, pl.ds(off, _L)] = (
                accb[t, pl.ds(off, _L)] + g[t] * rowb[t, pl.ds(off, _L)])
        return 0

    lax.fori_loop(0, nchunks, acc_body, 0)


def _sc_deep_body(x_hbm, w1_hbm, w2_hbm, part_hbm, nodes_hbm, out_hbm,
                  xbA, xbB, accbA, accbB, rowbA, rowbB,
                  idxA0, idxA1, idxB0, idxB1, semA, semB):
    wid = lax.axis_index("s") * _NC + lax.axis_index("c")
    tok_per_w = x_hbm.shape[0] // _NW
    n_pairs = tok_per_w // (2 * _G)

    def pair(pi, _):
        baseA = wid * tok_per_w + pi * (2 * _G)
        baseB = baseA + _G
        pltpu.sync_copy(x_hbm.at[pl.ds(baseA, _G)], xbA)
        pltpu.sync_copy(part_hbm.at[pl.ds(baseA, _G)], accbA)
        pltpu.sync_copy(nodes_hbm.at[pl.ds(baseA, _G)], idxA0)
        pltpu.sync_copy(x_hbm.at[pl.ds(baseB, _G)], xbB)
        pltpu.sync_copy(part_hbm.at[pl.ds(baseB, _G)], accbB)
        pltpu.sync_copy(nodes_hbm.at[pl.ds(baseB, _G)], idxB0)
        ibA = (idxA0, idxA1)
        ibB = (idxB0, idxB1)
        cpA = pltpu.async_copy(w1_hbm.at[idxA0], rowbA, semA)
        cpB = pltpu.async_copy(w1_hbm.at[idxB0], rowbB, semB)
        for lvl in range(_N_DEEP):
            curA, nxtA = ibA[lvl % 2], ibA[(lvl + 1) % 2]
            curB, nxtB = ibB[lvl % 2], ibB[(lvl + 1) % 2]
            cpA.wait()
            lgA = _sc_dots(xbA, rowbA)
            gA = _sc_gelu(lgA)
            nxtA[...] = 2 * curA[...] + 1 + (lgA >= 0.0).astype(jnp.int32)
            cpA = pltpu.async_copy(w2_hbm.at[curA], rowbA, semA)
            cpB.wait()
            lgB = _sc_dots(xbB, rowbB)
            gB = _sc_gelu(lgB)
            nxtB[...] = 2 * curB[...] + 1 + (lgB >= 0.0).astype(jnp.int32)
            cpB = pltpu.async_copy(w2_hbm.at[curB], rowbB, semB)
            cpA.wait()
            _sc_accum(accbA, rowbA, gA)
            if lvl < _N_DEEP - 1:
                cpA = pltpu.async_copy(w1_hbm.at[nxtA], rowbA, semA)
            cpB.wait()
            _sc_accum(accbB, rowbB, gB)
            if lvl < _N_DEEP - 1:
                cpB = pltpu.async_copy(w1_hbm.at[nxtB], rowbB, semB)
        pltpu.sync_copy(accbA, out_hbm.at[pl.ds(baseA, _G)])
        pltpu.sync_copy(accbB, out_hbm.at[pl.ds(baseB, _G)])
        return 0

    lax.fori_loop(0, n_pairs, pair, 0)


def _sc_deep(x, w1s, w2s, partial, nodes7):
    b = x.shape[0]
    fn = pl.kernel(
        _sc_deep_body,
        mesh=plsc.VectorSubcoreMesh(core_axis_name="c", subcore_axis_name="s"),
        compiler_params=pltpu.CompilerParams(needs_layout_passes=False),
        out_type=jax.ShapeDtypeStruct((b, _D), jnp.float32),
        scratch_types=[
            pltpu.VMEM((_G, _D), jnp.float32),
            pltpu.VMEM((_G, _D), jnp.float32),
            pltpu.VMEM((_G, _D), jnp.float32),
            pltpu.VMEM((_G, _D), jnp.float32),
            pltpu.VMEM((_G, _D), jnp.float32),
            pltpu.VMEM((_G, _D), jnp.float32),
            pltpu.VMEM((_G,), jnp.int32),
            pltpu.VMEM((_G,), jnp.int32),
            pltpu.VMEM((_G,), jnp.int32),
            pltpu.VMEM((_G,), jnp.int32),
            pltpu.SemaphoreType.DMA,
            pltpu.SemaphoreType.DMA,
        ],
    )
    return fn(x, w1s, w2s, partial, nodes7)


def kernel(x, w1s, w2s):
    w1t = jnp.concatenate(
        [w1s[:_N_TOP], jnp.zeros((_N_TOP_PAD - _N_TOP, _D), jnp.float32)], 0)
    w2t = jnp.concatenate(
        [w2s[:_N_TOP], jnp.zeros((_N_TOP_PAD - _N_TOP, _D), jnp.float32)], 0)
    partial, nodes7 = _tc_shallow(x, w1t, w2t)
    out = _sc_deep(x, w1s, w2s, partial, nodes7)
    return out[:, None, :]
